# baseline (speedup 1.0000x reference)
import functools

import jax
import jax.numpy as jnp
from jax.experimental import pallas as pl
from jax.experimental.pallas import tpu as pltpu

_TOP_K = 2
_BLOCK_T = 512


def _router_block(x_ref, wt_ref, idx_ref, w_ref):
    x = x_ref[...]
    wt = wt_ref[...]
    logits = jnp.dot(x, wt, preferred_element_type=jnp.float32)
    m = jnp.max(logits, axis=-1, keepdims=True)
    e = jnp.exp(logits - m)
    s = e / jnp.sum(e, axis=-1, keepdims=True)
    n_e = s.shape[-1]
    lane = jax.lax.broadcasted_iota(jnp.int32, s.shape, 1)
    m1 = jnp.max(s, axis=-1, keepdims=True)
    i1 = jnp.min(jnp.where(s == m1, lane, n_e), axis=-1, keepdims=True)
    s2 = jnp.where(lane == i1, -1.0, s)
    m2 = jnp.max(s2, axis=-1, keepdims=True)
    i2 = jnp.min(jnp.where(s2 == m2, lane, n_e), axis=-1, keepdims=True)
    denom = m1 + m2 + 1e-20
    idx_ref[...] = jnp.concatenate([i1, i2], axis=-1)
    w_ref[...] = jnp.concatenate([m1 / denom, m2 / denom], axis=-1)


@jax.jit
def kernel(hidden_states, weight):
    bsz, seq_len, h = hidden_states.shape
    n_tok = bsz * seq_len
    n_exp = weight.shape[0]
    x = hidden_states.reshape(n_tok, h).astype(jnp.float32)
    wt = weight.astype(jnp.float32).T

    grid = (n_tok // _BLOCK_T,)
    topk_idx, topk_w = pl.pallas_call(
        _router_block,
        grid=grid,
        in_specs=[
            pl.BlockSpec((_BLOCK_T, h), lambda i: (i, 0)),
            pl.BlockSpec((h, n_exp), lambda i: (0, 0)),
        ],
        out_specs=[
            pl.BlockSpec((_BLOCK_T, _TOP_K), lambda i: (i, 0)),
            pl.BlockSpec((_BLOCK_T, _TOP_K), lambda i: (i, 0)),
        ],
        out_shape=[
            jax.ShapeDtypeStruct((n_tok, _TOP_K), jnp.int32),
            jax.ShapeDtypeStruct((n_tok, _TOP_K), jnp.float32),
        ],
        compiler_params=pltpu.CompilerParams(
            dimension_semantics=("arbitrary",),
        ),
    )(x, wt)
    return (topk_idx, topk_w)


# speedup vs baseline: 1.2833x; 1.2833x over previous
import functools

import jax
import jax.numpy as jnp
from jax.experimental import pallas as pl
from jax.experimental.pallas import tpu as pltpu

_TOP_K = 2
_BLOCK_T = 2048


def _router_block(x_ref, wt_ref, idx_ref, w_ref):
    x = x_ref[...]
    wt = wt_ref[...]
    logits = jnp.dot(x, wt, preferred_element_type=jnp.float32)
    m = jnp.max(logits, axis=-1, keepdims=True)
    e = jnp.exp(logits - m)
    s = e / jnp.sum(e, axis=-1, keepdims=True)
    n_e = s.shape[-1]
    lane = jax.lax.broadcasted_iota(jnp.int32, s.shape, 1)
    m1 = jnp.max(s, axis=-1, keepdims=True)
    i1 = jnp.min(jnp.where(s == m1, lane, n_e), axis=-1, keepdims=True)
    s2 = jnp.where(lane == i1, -1.0, s)
    m2 = jnp.max(s2, axis=-1, keepdims=True)
    i2 = jnp.min(jnp.where(s2 == m2, lane, n_e), axis=-1, keepdims=True)
    denom = m1 + m2 + 1e-20
    idx_ref[...] = jnp.concatenate([i1, i2], axis=-1)
    w_ref[...] = jnp.concatenate([m1 / denom, m2 / denom], axis=-1)


@jax.jit
def kernel(hidden_states, weight):
    bsz, seq_len, h = hidden_states.shape
    n_tok = bsz * seq_len
    n_exp = weight.shape[0]
    x = hidden_states.reshape(n_tok, h).astype(jnp.float32)
    wt = weight.astype(jnp.float32).T

    grid = (n_tok // _BLOCK_T,)
    topk_idx, topk_w = pl.pallas_call(
        _router_block,
        grid=grid,
        in_specs=[
            pl.BlockSpec((_BLOCK_T, h), lambda i: (i, 0)),
            pl.BlockSpec((h, n_exp), lambda i: (0, 0)),
        ],
        out_specs=[
            pl.BlockSpec((_BLOCK_T, _TOP_K), lambda i: (i, 0)),
            pl.BlockSpec((_BLOCK_T, _TOP_K), lambda i: (i, 0)),
        ],
        out_shape=[
            jax.ShapeDtypeStruct((n_tok, _TOP_K), jnp.int32),
            jax.ShapeDtypeStruct((n_tok, _TOP_K), jnp.float32),
        ],
        compiler_params=pltpu.CompilerParams(
            dimension_semantics=("arbitrary",),
        ),
    )(x, wt)
    return (topk_idx, topk_w)


# speedup vs baseline: 1.7705x; 1.3796x over previous
import functools

import jax
import jax.numpy as jnp
from jax.experimental import pallas as pl
from jax.experimental.pallas import tpu as pltpu

_TOP_K = 2
_BLOCK_T = 2048


def _router_block(x_ref, wt_ref, idx_ref, w_ref):
    x = x_ref[...]
    wt = wt_ref[...]
    logits = jnp.dot(x, wt, preferred_element_type=jnp.float32)
    lt = logits.T
    n_e = lt.shape[0]
    rows = [lt[e] for e in range(n_e)]

    m1 = rows[0]
    for e in range(1, n_e):
        m1 = jnp.maximum(m1, rows[e])
    i1 = jnp.full_like(m1, n_e - 1, dtype=jnp.int32)
    for e in range(n_e - 2, -1, -1):
        i1 = jnp.where(rows[e] == m1, e, i1)

    neg = jnp.float32(-3.0e38)
    rows2 = [jnp.where(i1 == e, neg, rows[e]) for e in range(n_e)]
    m2 = rows2[0]
    for e in range(1, n_e):
        m2 = jnp.maximum(m2, rows2[e])
    i2 = jnp.full_like(m1, n_e - 1, dtype=jnp.int32)
    for e in range(n_e - 2, -1, -1):
        i2 = jnp.where(rows2[e] == m2, e, i2)

    d = jnp.exp(m2 - m1)
    r = 1.0 / (1.0 + d)
    idx_ref[...] = jnp.stack([i1, i2], axis=0)
    w_ref[...] = jnp.stack([r, d * r], axis=0)


@jax.jit
def kernel(hidden_states, weight):
    bsz, seq_len, h = hidden_states.shape
    n_tok = bsz * seq_len
    n_exp = weight.shape[0]
    x = hidden_states.reshape(n_tok, h).astype(jnp.float32)
    wt = weight.astype(jnp.float32).T

    grid = (n_tok // _BLOCK_T,)
    idx_t, w_t = pl.pallas_call(
        _router_block,
        grid=grid,
        in_specs=[
            pl.BlockSpec((_BLOCK_T, h), lambda i: (i, 0)),
            pl.BlockSpec((h, n_exp), lambda i: (0, 0)),
        ],
        out_specs=[
            pl.BlockSpec((_TOP_K, _BLOCK_T), lambda i: (0, i)),
            pl.BlockSpec((_TOP_K, _BLOCK_T), lambda i: (0, i)),
        ],
        out_shape=[
            jax.ShapeDtypeStruct((_TOP_K, n_tok), jnp.int32),
            jax.ShapeDtypeStruct((_TOP_K, n_tok), jnp.float32),
        ],
        compiler_params=pltpu.CompilerParams(
            dimension_semantics=("arbitrary",),
        ),
    )(x, wt)
    return (idx_t.T, w_t.T)
